# Initial kernel scaffold; baseline (speedup 1.0000x reference)
import jax
import jax.numpy as jnp
from jax.experimental import pallas as pl


def kernel(x, tables):
    raise NotImplementedError("write your pallas kernel here")


# speedup vs baseline: 1.3582x; 1.3582x over previous
import functools

import jax
import jax.numpy as jnp
from jax import lax
from jax.experimental import pallas as pl
from jax.experimental.pallas import tpu as pltpu
from jax.experimental.pallas import tpu_sc as plsc

BATCH = 16384
NF = 26
D = 16
VOCAB = 100000
XW = 2 * NF
OUT_W = NF * D + NF

NC = 2
NS = 16
NW = NC * NS
B_PER_W = BATCH // NW
BC = 128
N_CHUNK = B_PER_W // BC
L = 16


def _body(x_hbm, tab_hbm, out_hbm, xv, idx_v, emb_v, cont_a, gsem, osem):
    wid = lax.axis_index("s") * NC + lax.axis_index("c")
    iota = lax.iota(jnp.int32, L)

    def chunk(c, carry):
        base = wid * B_PER_W + c * BC

        pltpu.sync_copy(x_hbm.at[pl.ds(base * XW, BC * XW)], xv)

        for j in range(NF * BC // L):
            f = j // (BC // L)
            b0 = (j % (BC // L)) * L
            src = (b0 + iota) * XW + f
            vals = plsc.load_gather(xv, [src])
            idx_v[pl.ds(j * L, L)] = vals + f * VOCAB

        gathers = [
            pltpu.async_copy(
                tab_hbm.at[idx_v.at[pl.ds(f * BC, BC)]],
                emb_v.at[pl.ds(f * BC, BC), :],
                gsem,
            )
            for f in range(NF)
        ]

        for j in range(NF * BC // L):
            p = j * L + iota
            b = p // NF
            col = p - b * NF
            vals = plsc.load_gather(xv, [p + b * NF + NF])
            plsc.store_scatter(cont_a, [b, col], vals.astype(jnp.float32))

        for g in gathers:
            g.wait()

        outs = [
            pltpu.async_copy(
                emb_v.at[pl.ds(f * BC, BC), :],
                out_hbm.at[pl.ds(base, BC), pl.ds(f * D, D)],
                osem,
            )
            for f in range(NF)
        ]
        outs.append(
            pltpu.async_copy(
                cont_a, out_hbm.at[pl.ds(base, BC), pl.ds(NF * D, NF)], osem
            )
        )
        for o in outs:
            o.wait()
        return carry

    lax.fori_loop(0, N_CHUNK, chunk, 0)


@jax.jit
def _emb_lookup(x_flat, tab):
    run = pl.kernel(
        _body,
        out_type=jax.ShapeDtypeStruct((BATCH, OUT_W), jnp.float32),
        mesh=plsc.VectorSubcoreMesh(
            core_axis_name="c", subcore_axis_name="s", num_cores=NC,
            num_subcores=NS,
        ),
        scratch_types=[
            pltpu.VMEM((BC * XW,), jnp.int32),
            pltpu.VMEM((NF * BC,), jnp.int32),
            pltpu.VMEM((NF * BC, D), jnp.float32),
            pltpu.VMEM((BC, NF), jnp.float32),
            pltpu.SemaphoreType.DMA,
            pltpu.SemaphoreType.DMA,
        ],
        compiler_params=pltpu.CompilerParams(
            use_tc_tiling_on_sc=False, needs_layout_passes=False
        ),
    )
    return run(x_flat, tab)


def kernel(x, tables):
    return _emb_lookup(x.reshape(-1), tables.reshape(NF * VOCAB, D))
